# Initial kernel scaffold; baseline (speedup 1.0000x reference)
import jax
import jax.numpy as jnp
from jax.experimental import pallas as pl


def kernel(pc1, est_flow, pc2):
    raise NotImplementedError("write your pallas kernel here")


# speedup vs baseline: 9.6842x; 9.6842x over previous
import functools

import jax
import jax.numpy as jnp
from jax import lax
from jax.experimental import pallas as pl
from jax.experimental.pallas import tpu as pltpu
from jax.experimental.pallas import tpu_sc as plsc

_N = 16384
_K = 12
_MAX_RADIUS = 2.0
_M = 256
_CHUNK = 2048
_GROUP = 128
_NW = 32
_PW = _N // _NW
_LANES = 16

_IMAX = 0x7FFFFFFF
_IDXMASK = 0x3FFF


def _tc_knn_body(q_ref, pt_ref, out_ref):
    q = q_ref[...]
    qsq = jnp.sum(q * q, axis=1, keepdims=True)
    cands = []
    for c in range(_N // _CHUNK):
        ptc = pt_ref[:, c * _CHUNK:(c + 1) * _CHUNK]
        qp = lax.dot_general(q.astype(jnp.bfloat16), ptc.astype(jnp.bfloat16),
                             (((1,), (0,)), ((), ())),
                             preferred_element_type=jnp.float32)
        psq = jnp.sum(ptc * ptc, axis=0, keepdims=True)
        d = qsq - 2.0 * qp + psq
        b = lax.bitcast_convert_type(d, jnp.int32)
        key = (b ^ ((b >> 31) & 0x7FFFFFFF)) + 0x2000
        col = lax.broadcasted_iota(jnp.int32, (_M, _CHUNK), 1) + c * _CHUNK
        key = (key & ~_IDXMASK) | col
        k3 = key.reshape(_M, _CHUNK // _GROUP, _GROUP)
        m1 = jnp.min(k3, axis=2)
        k3 = jnp.where(k3 == m1[:, :, None], _IMAX, k3)
        m2 = jnp.min(k3, axis=2)
        cands.append(m1)
        cands.append(m2)
    cand = jnp.concatenate(cands, axis=1)
    ranks = []
    for _ in range(_K):
        m = jnp.min(cand, axis=1, keepdims=True)
        ranks.append(m)
        cand = jnp.where(cand == m, _IMAX, cand)
    rk = jnp.concatenate(ranks, axis=1)
    idx = rk & _IDXMASK
    dm = rk - idx
    dtr = lax.bitcast_convert_type(dm ^ ((dm >> 31) & 0x7FFFFFFF), jnp.float32)
    idx0 = idx[:, 0:1]
    idxm = jnp.where(dtr > _MAX_RADIUS, idx0, idx)
    out_ref[...] = jnp.concatenate([idxm, idx0, idx0, idx0, idx0], axis=1)


def _tc_knn(p_rows, pt):
    return pl.pallas_call(
        _tc_knn_body,
        grid=(_N // _M,),
        in_specs=[
            pl.BlockSpec((_M, 8), lambda i: (i, 0)),
            pl.BlockSpec((8, _N), lambda i: (0, 0)),
        ],
        out_specs=pl.BlockSpec((_M, 16), lambda i: (i, 0)),
        out_shape=jax.ShapeDtypeStruct((_N, 16), jnp.int32),
    )(p_rows, pt)


@functools.cache
def _sc_smooth_kernel():
    return functools.partial(
        pl.kernel,
        mesh=plsc.VectorSubcoreMesh(core_axis_name="c", subcore_axis_name="s"),
        compiler_params=pltpu.CompilerParams(needs_layout_passes=False),
        out_type=jax.ShapeDtypeStruct((_NW, _LANES), jnp.float32),
        scratch_types=[
            pltpu.VMEM((_N,), jnp.float32),
            pltpu.VMEM((_N,), jnp.float32),
            pltpu.VMEM((_N,), jnp.float32),
            pltpu.VMEM((_PW * 16,), jnp.int32),
            pltpu.VMEM((_LANES,), jnp.float32),
        ],
    )(_sc_smooth_body)


def _sc_smooth_body(fx_hbm, fy_hbm, fz_hbm, nn_hbm, out_hbm,
                    fx_v, fy_v, fz_v, idx_v, acc_v):
    wid = lax.axis_index("s") * 2 + lax.axis_index("c")
    base = wid * _PW
    pltpu.sync_copy(fx_hbm, fx_v)
    pltpu.sync_copy(fy_hbm, fy_v)
    pltpu.sync_copy(fz_hbm, fz_v)
    pltpu.sync_copy(nn_hbm.at[pl.ds(base * 16, _PW * 16)], idx_v)

    def step(v, acc):
        rows = lax.iota(jnp.int32, _LANES) + v * _LANES
        ox = fx_v[pl.ds(base + v * _LANES, _LANES)]
        oy = fy_v[pl.ds(base + v * _LANES, _LANES)]
        oz = fz_v[pl.ds(base + v * _LANES, _LANES)]
        for k in range(1, _K):
            nnv = plsc.load_gather(idx_v, [rows * 16 + k])
            gx = plsc.load_gather(fx_v, [nnv])
            gy = plsc.load_gather(fy_v, [nnv])
            gz = plsc.load_gather(fz_v, [nnv])
            acc = acc + jnp.abs(gx - ox) + jnp.abs(gy - oy) + jnp.abs(gz - oz)
        return acc

    acc = lax.fori_loop(0, _PW // _LANES, step,
                        jnp.zeros((_LANES,), jnp.float32))
    acc_v[...] = acc
    pltpu.sync_copy(acc_v, out_hbm.at[wid])


def kernel(pc1, est_flow, pc2):
    p = pc1[0]
    p_rows = jnp.pad(p, ((0, 0), (0, 5)))
    pt = p_rows.T
    nn = _tc_knn(p_rows, pt)
    flow = est_flow[0]
    partial = _sc_smooth_kernel()(flow[:, 0], flow[:, 1], flow[:, 2],
                                  nn.reshape(-1))
    return jnp.sum(partial) / jnp.float32((_K - 1) * _N)


# speedup vs baseline: 14.3028x; 1.4769x over previous
import functools

import jax
import jax.numpy as jnp
from jax import lax
from jax.experimental import pallas as pl
from jax.experimental.pallas import tpu as pltpu
from jax.experimental.pallas import tpu_sc as plsc

_N = 16384
_K = 12
_MAX_RADIUS = 2.0
_M = 256
_CHUNK = 2048
_GROUP = 64
_NW = 32
_PW = _N // _NW
_LANES = 16

_IMAX = 0x7FFFFFFF
_IDXMASK = 0x3FFF


def _tc_knn_body(q_ref, qsq_ref, ptm2_ref, psq2_ref, out_ref):
    q = q_ref[...]
    qsq = qsq_ref[...]
    lane6 = lax.broadcasted_iota(jnp.int32, (_M, _CHUNK), 1) & (_GROUP - 1)
    cands = []
    for c in range(_N // _CHUNK):
        sl = pl.ds(c * _CHUNK, _CHUNK)
        ptc = ptm2_ref[:, sl]
        psq2 = psq2_ref[:, sl]
        qp2 = lax.dot_general(q, ptc, (((1,), (0,)), ((), ())),
                              preferred_element_type=jnp.float32)
        d2 = (qsq + qp2) + psq2
        kb = lax.bitcast_convert_type(d2, jnp.int32)
        kf = lax.bitcast_convert_type((kb & ~(_GROUP - 1)) | lane6,
                                      jnp.float32)
        k3 = kf.reshape(_M, _CHUNK // _GROUP, _GROUP)
        cands.append(jnp.min(k3, axis=2))
    cand = jnp.concatenate(cands, axis=1)
    ib = lax.bitcast_convert_type(cand, jnp.int32)
    lane = ib & (_GROUP - 1)
    dv = lax.bitcast_convert_type(ib - lane, jnp.float32) - 2.0
    b2 = lax.bitcast_convert_type(dv, jnp.int32)
    b2 = b2 ^ ((b2 >> 31) & 0x7FFFFFFF)
    col8 = lax.broadcasted_iota(jnp.int32, (_M, _N // _GROUP), 1)
    cand = (((b2 + 0x2000) & ~_IDXMASK) | (col8 * _GROUP)) | lane
    ranks = []
    for _ in range(_K):
        m = jnp.min(cand, axis=1, keepdims=True)
        ranks.append(m)
        cand = jnp.where(cand == m, _IMAX, cand)
    rk = jnp.concatenate(ranks, axis=1)
    idx = rk & _IDXMASK
    dm = rk - idx
    dtr = lax.bitcast_convert_type(dm ^ ((dm >> 31) & 0x7FFFFFFF), jnp.float32)
    idx0 = idx[:, 0:1]
    idxm = jnp.where(dtr > _MAX_RADIUS, idx0, idx)
    out_ref[...] = jnp.concatenate([idxm, idx0, idx0, idx0, idx0], axis=1)


def _tc_knn(q_bf16, qsq, ptm2, psq2):
    return pl.pallas_call(
        _tc_knn_body,
        grid=(_N // _M,),
        in_specs=[
            pl.BlockSpec((_M, 16), lambda i: (i, 0)),
            pl.BlockSpec((_M, 1), lambda i: (i, 0)),
            pl.BlockSpec((16, _N), lambda i: (0, 0)),
            pl.BlockSpec((1, _N), lambda i: (0, 0)),
        ],
        out_specs=pl.BlockSpec((_M, 16), lambda i: (i, 0)),
        out_shape=jax.ShapeDtypeStruct((_N, 16), jnp.int32),
    )(q_bf16, qsq, ptm2, psq2)


@functools.cache
def _sc_smooth_kernel():
    return functools.partial(
        pl.kernel,
        mesh=plsc.VectorSubcoreMesh(core_axis_name="c", subcore_axis_name="s"),
        compiler_params=pltpu.CompilerParams(needs_layout_passes=False),
        out_type=jax.ShapeDtypeStruct((_NW, _LANES), jnp.float32),
        scratch_types=[
            pltpu.VMEM((_N,), jnp.float32),
            pltpu.VMEM((_N,), jnp.float32),
            pltpu.VMEM((_N,), jnp.float32),
            pltpu.VMEM((_PW * 16,), jnp.int32),
            pltpu.VMEM((_LANES,), jnp.float32),
        ],
    )(_sc_smooth_body)


def _sc_smooth_body(fx_hbm, fy_hbm, fz_hbm, nn_hbm, out_hbm,
                    fx_v, fy_v, fz_v, idx_v, acc_v):
    wid = lax.axis_index("s") * 2 + lax.axis_index("c")
    base = wid * _PW
    pltpu.sync_copy(fx_hbm, fx_v)
    pltpu.sync_copy(fy_hbm, fy_v)
    pltpu.sync_copy(fz_hbm, fz_v)
    pltpu.sync_copy(nn_hbm.at[pl.ds(base * 16, _PW * 16)], idx_v)

    def step(v, acc):
        rows = lax.iota(jnp.int32, _LANES) + v * _LANES
        ox = fx_v[pl.ds(base + v * _LANES, _LANES)]
        oy = fy_v[pl.ds(base + v * _LANES, _LANES)]
        oz = fz_v[pl.ds(base + v * _LANES, _LANES)]
        for k in range(1, _K):
            nnv = plsc.load_gather(idx_v, [rows * 16 + k])
            gx = plsc.load_gather(fx_v, [nnv])
            gy = plsc.load_gather(fy_v, [nnv])
            gz = plsc.load_gather(fz_v, [nnv])
            acc = acc + jnp.abs(gx - ox) + jnp.abs(gy - oy) + jnp.abs(gz - oz)
        return acc

    acc = lax.fori_loop(0, _PW // _LANES, step,
                        jnp.zeros((_LANES,), jnp.float32))
    acc_v[...] = acc
    pltpu.sync_copy(acc_v, out_hbm.at[wid])


def kernel(pc1, est_flow, pc2):
    p = pc1[0]
    p_rows = jnp.pad(p, ((0, 0), (0, 13)))
    q_bf16 = p_rows.astype(jnp.bfloat16)
    ptm2 = (-2.0 * p_rows).T.astype(jnp.bfloat16)
    sq = jnp.sum(p * p, axis=1, keepdims=True)
    nn = _tc_knn(q_bf16, sq, ptm2, sq.T + 2.0)
    flow = est_flow[0]
    partial = _sc_smooth_kernel()(flow[:, 0], flow[:, 1], flow[:, 2],
                                  nn.reshape(-1))
    return jnp.sum(partial) / jnp.float32((_K - 1) * _N)


# speedup vs baseline: 15.3132x; 1.0706x over previous
import functools

import jax
import jax.numpy as jnp
from jax import lax
from jax.experimental import pallas as pl
from jax.experimental.pallas import tpu as pltpu
from jax.experimental.pallas import tpu_sc as plsc

_N = 16384
_K = 12
_MAX_RADIUS = 2.0
_M = 256
_CHUNK = 2048
_GROUP = 64
_NW = 32
_PW = _N // _NW
_LANES = 16

_IMAX = 0x7FFFFFFF
_IDXMASK = 0x3FFF


def _tc_knn_body(q_ref, ptm2_ref, out_ref):
    q = q_ref[...]
    lane6 = lax.broadcasted_iota(jnp.int32, (_M, _CHUNK), 1) & (_GROUP - 1)
    cands = []
    for c in range(_N // _CHUNK):
        sl = pl.ds(c * _CHUNK, _CHUNK)
        ptc = ptm2_ref[:, sl]
        d2 = lax.dot_general(q, ptc, (((1,), (0,)), ((), ())),
                             preferred_element_type=jnp.float32)
        kb = lax.bitcast_convert_type(d2, jnp.int32)
        kf = lax.bitcast_convert_type((kb & ~(_GROUP - 1)) | lane6,
                                      jnp.float32)
        k3 = kf.reshape(_M, _CHUNK // _GROUP, _GROUP)
        cands.append(jnp.min(k3, axis=2))
    cand = jnp.concatenate(cands, axis=1)
    ib = lax.bitcast_convert_type(cand, jnp.int32)
    lane = ib & (_GROUP - 1)
    dv = lax.bitcast_convert_type(ib - lane, jnp.float32) - 2.0
    b2 = lax.bitcast_convert_type(dv, jnp.int32)
    b2 = b2 ^ ((b2 >> 31) & 0x7FFFFFFF)
    col8 = lax.broadcasted_iota(jnp.int32, (_M, _N // _GROUP), 1)
    cand = (((b2 + 0x2000) & ~_IDXMASK) | (col8 * _GROUP)) | lane
    ranks = []
    for _ in range(_K):
        m = jnp.min(cand, axis=1, keepdims=True)
        ranks.append(m)
        cand = jnp.where(cand == m, _IMAX, cand)
    rk = jnp.concatenate(ranks, axis=1)
    idx = rk & _IDXMASK
    dm = rk - idx
    dtr = lax.bitcast_convert_type(dm ^ ((dm >> 31) & 0x7FFFFFFF), jnp.float32)
    idx0 = idx[:, 0:1]
    idxm = jnp.where(dtr > _MAX_RADIUS, idx0, idx)
    out_ref[...] = jnp.concatenate([idxm, idx0, idx0, idx0, idx0], axis=1)


def _tc_knn(q_bf16, ptm2):
    return pl.pallas_call(
        _tc_knn_body,
        grid=(_N // _M,),
        in_specs=[
            pl.BlockSpec((_M, 16), lambda i: (i, 0)),
            pl.BlockSpec((16, _N), lambda i: (0, 0)),
        ],
        out_specs=pl.BlockSpec((_M, 16), lambda i: (i, 0)),
        out_shape=jax.ShapeDtypeStruct((_N, 16), jnp.int32),
    )(q_bf16, ptm2)


@functools.cache
def _sc_smooth_kernel():
    return functools.partial(
        pl.kernel,
        mesh=plsc.VectorSubcoreMesh(core_axis_name="c", subcore_axis_name="s"),
        compiler_params=pltpu.CompilerParams(needs_layout_passes=False),
        out_type=jax.ShapeDtypeStruct((_NW, _LANES), jnp.float32),
        scratch_types=[
            pltpu.VMEM((_N,), jnp.float32),
            pltpu.VMEM((_N,), jnp.float32),
            pltpu.VMEM((_N,), jnp.float32),
            pltpu.VMEM((_PW * 16,), jnp.int32),
            pltpu.VMEM((_LANES,), jnp.float32),
        ],
    )(_sc_smooth_body)


def _sc_smooth_body(fx_hbm, fy_hbm, fz_hbm, nn_hbm, out_hbm,
                    fx_v, fy_v, fz_v, idx_v, acc_v):
    wid = lax.axis_index("s") * 2 + lax.axis_index("c")
    base = wid * _PW
    pltpu.sync_copy(fx_hbm, fx_v)
    pltpu.sync_copy(fy_hbm, fy_v)
    pltpu.sync_copy(fz_hbm, fz_v)
    pltpu.sync_copy(nn_hbm.at[pl.ds(base * 16, _PW * 16)], idx_v)

    def step(v, acc):
        rows = lax.iota(jnp.int32, _LANES) + v * _LANES
        ox = fx_v[pl.ds(base + v * _LANES, _LANES)]
        oy = fy_v[pl.ds(base + v * _LANES, _LANES)]
        oz = fz_v[pl.ds(base + v * _LANES, _LANES)]
        for k in range(1, _K):
            nnv = plsc.load_gather(idx_v, [rows * 16 + k])
            gx = plsc.load_gather(fx_v, [nnv])
            gy = plsc.load_gather(fy_v, [nnv])
            gz = plsc.load_gather(fz_v, [nnv])
            acc = acc + jnp.abs(gx - ox) + jnp.abs(gy - oy) + jnp.abs(gz - oz)
        return acc

    acc = lax.fori_loop(0, _PW // _LANES, step,
                        jnp.zeros((_LANES,), jnp.float32))
    acc_v[...] = acc
    pltpu.sync_copy(acc_v, out_hbm.at[wid])


def kernel(pc1, est_flow, pc2):
    p = pc1[0]
    pb = p.astype(jnp.bfloat16)
    sq = jnp.sum(p * p, axis=1, keepdims=True)
    one = jnp.ones((_N, 1), jnp.bfloat16)
    zero3 = jnp.zeros((_N, 9), jnp.bfloat16)
    qh = sq.astype(jnp.bfloat16)
    ql = (sq - qh.astype(jnp.float32)).astype(jnp.bfloat16)
    psq2 = sq + 2.0
    ph = psq2.astype(jnp.bfloat16)
    pl_ = (psq2 - ph.astype(jnp.float32)).astype(jnp.bfloat16)
    q_bf16 = jnp.concatenate([pb, one, qh, ql, one, zero3], axis=1)
    ptm2 = jnp.concatenate([(-2.0 * p).astype(jnp.bfloat16), ph, one, one,
                            pl_, zero3], axis=1).T
    nn = _tc_knn(q_bf16, ptm2)
    flow = est_flow[0]
    partial = _sc_smooth_kernel()(flow[:, 0], flow[:, 1], flow[:, 2],
                                  nn.reshape(-1))
    return jnp.sum(partial) / jnp.float32((_K - 1) * _N)


# speedup vs baseline: 16.0346x; 1.0471x over previous
import functools

import jax
import jax.numpy as jnp
from jax import lax
from jax.experimental import pallas as pl
from jax.experimental.pallas import tpu as pltpu
from jax.experimental.pallas import tpu_sc as plsc

_N = 16384
_K = 12
_MAX_RADIUS = 2.0
_M = 512
_CHUNK = 2048
_GROUP = 64
_NW = 32
_PW = _N // _NW
_LANES = 16

_IMAX = 0x7FFFFFFF
_IDXMASK = 0x3FFF


def _tc_knn_body(q_ref, qsq_ref, ptm2_ref, psq2_ref, out_ref):
    q = q_ref[...]
    qsq = qsq_ref[...]
    lane6 = lax.broadcasted_iota(jnp.int32, (_M, _CHUNK), 1) & (_GROUP - 1)
    cands = []
    for c in range(_N // _CHUNK):
        sl = pl.ds(c * _CHUNK, _CHUNK)
        ptc = ptm2_ref[:, sl]
        psq2 = psq2_ref[:, sl]
        qp2 = lax.dot_general(q, ptc, (((1,), (0,)), ((), ())),
                              preferred_element_type=jnp.float32)
        d2 = (qsq + qp2) + psq2
        kb = lax.bitcast_convert_type(d2, jnp.int32)
        kf = lax.bitcast_convert_type((kb & ~(_GROUP - 1)) | lane6,
                                      jnp.float32)
        k3 = kf.reshape(_M, _CHUNK // _GROUP, _GROUP)
        cands.append(jnp.min(k3, axis=2))
    cand = jnp.concatenate(cands, axis=1)
    ib = lax.bitcast_convert_type(cand, jnp.int32)
    lane = ib & (_GROUP - 1)
    dv = lax.bitcast_convert_type(ib - lane, jnp.float32) - 2.0
    b2 = lax.bitcast_convert_type(dv, jnp.int32)
    b2 = b2 ^ ((b2 >> 31) & 0x7FFFFFFF)
    col8 = lax.broadcasted_iota(jnp.int32, (_M, _N // _GROUP), 1)
    cand = (((b2 + 0x2000) & ~_IDXMASK) | (col8 * _GROUP)) | lane
    ranks = []
    for _ in range(_K):
        m = jnp.min(cand, axis=1, keepdims=True)
        ranks.append(m)
        cand = jnp.where(cand == m, _IMAX, cand)
    rk = jnp.concatenate(ranks, axis=1)
    idx = rk & _IDXMASK
    dm = rk - idx
    dtr = lax.bitcast_convert_type(dm ^ ((dm >> 31) & 0x7FFFFFFF), jnp.float32)
    idx0 = idx[:, 0:1]
    idxm = jnp.where(dtr > _MAX_RADIUS, idx0, idx)
    out_ref[...] = jnp.concatenate([idxm, idx0, idx0, idx0, idx0], axis=1)


def _tc_knn(q_bf16, qsq, ptm2, psq2):
    return pl.pallas_call(
        _tc_knn_body,
        grid=(_N // _M,),
        in_specs=[
            pl.BlockSpec((_M, 16), lambda i: (i, 0)),
            pl.BlockSpec((_M, 1), lambda i: (i, 0)),
            pl.BlockSpec((16, _N), lambda i: (0, 0)),
            pl.BlockSpec((1, _N), lambda i: (0, 0)),
        ],
        out_specs=pl.BlockSpec((_M, 16), lambda i: (i, 0)),
        out_shape=jax.ShapeDtypeStruct((_N, 16), jnp.int32),
    )(q_bf16, qsq, ptm2, psq2)


@functools.cache
def _sc_smooth_kernel():
    return functools.partial(
        pl.kernel,
        mesh=plsc.VectorSubcoreMesh(core_axis_name="c", subcore_axis_name="s"),
        compiler_params=pltpu.CompilerParams(needs_layout_passes=False),
        out_type=jax.ShapeDtypeStruct((_NW, _LANES), jnp.float32),
        scratch_types=[
            pltpu.VMEM((_N,), jnp.float32),
            pltpu.VMEM((_N,), jnp.float32),
            pltpu.VMEM((_N,), jnp.float32),
            pltpu.VMEM((_PW * 16,), jnp.int32),
            pltpu.VMEM((_LANES,), jnp.float32),
        ],
    )(_sc_smooth_body)


def _sc_smooth_body(fx_hbm, fy_hbm, fz_hbm, nn_hbm, out_hbm,
                    fx_v, fy_v, fz_v, idx_v, acc_v):
    wid = lax.axis_index("s") * 2 + lax.axis_index("c")
    base = wid * _PW
    pltpu.sync_copy(fx_hbm, fx_v)
    pltpu.sync_copy(fy_hbm, fy_v)
    pltpu.sync_copy(fz_hbm, fz_v)
    pltpu.sync_copy(nn_hbm.at[pl.ds(base * 16, _PW * 16)], idx_v)

    def step(v, acc):
        rows = lax.iota(jnp.int32, _LANES) + v * _LANES
        ox = fx_v[pl.ds(base + v * _LANES, _LANES)]
        oy = fy_v[pl.ds(base + v * _LANES, _LANES)]
        oz = fz_v[pl.ds(base + v * _LANES, _LANES)]
        for k in range(1, _K):
            nnv = plsc.load_gather(idx_v, [rows * 16 + k])
            gx = plsc.load_gather(fx_v, [nnv])
            gy = plsc.load_gather(fy_v, [nnv])
            gz = plsc.load_gather(fz_v, [nnv])
            acc = acc + jnp.abs(gx - ox) + jnp.abs(gy - oy) + jnp.abs(gz - oz)
        return acc

    acc = lax.fori_loop(0, _PW // _LANES, step,
                        jnp.zeros((_LANES,), jnp.float32))
    acc_v[...] = acc
    pltpu.sync_copy(acc_v, out_hbm.at[wid])


def kernel(pc1, est_flow, pc2):
    p = pc1[0]
    p_rows = jnp.pad(p, ((0, 0), (0, 13)))
    q_bf16 = p_rows.astype(jnp.bfloat16)
    ptm2 = (-2.0 * p_rows).T.astype(jnp.bfloat16)
    sq = jnp.sum(p * p, axis=1, keepdims=True)
    nn = _tc_knn(q_bf16, sq, ptm2, sq.T + 2.0)
    flow = est_flow[0]
    partial = _sc_smooth_kernel()(flow[:, 0], flow[:, 1], flow[:, 2],
                                  nn.reshape(-1))
    return jnp.sum(partial) / jnp.float32((_K - 1) * _N)
